# baseline (speedup 1.0000x reference)
import jax
import jax.numpy as jnp
from jax import lax
from jax.experimental import pallas as pl
from jax.experimental.pallas import tpu as pltpu
from jax.experimental.pallas import tpu_sc as plsc

NODES = 26
P = 32
IN_DIM = 128
H0 = 48
H1 = 60
G = 8
GP = G * P

NW = 32
XCHUNK = 16


def _sc_gather(gx_hbm, kt_hbm, idx_hbm, outx_hbm, outk_hbm,
               idx_v, xstage, kstage, sem):
    wid = lax.axis_index("s") * 2 + lax.axis_index("c")
    nrow = idx_v.shape[0]
    base = wid * nrow
    pltpu.sync_copy(idx_hbm.at[pl.ds(base, nrow)], idx_v)
    for c in range(nrow):
        pltpu.async_copy(kt_hbm.at[idx_v.at[c]], kstage, sem).wait()
        pltpu.sync_copy(
            kstage, outk_hbm.at[pl.ds((base + c) * XCHUNK, XCHUNK)])
    for c in range(nrow):
        pltpu.async_copy(gx_hbm.at[idx_v.at[c]], xstage, sem).wait()
        pltpu.sync_copy(
            xstage, outx_hbm.at[pl.ds((base + c) * XCHUNK, XCHUNK)])


def _sc_gather_call(graph_x, kernel_t, indices):
    B = indices.shape[0]
    idx2d = indices.astype(jnp.int32).reshape(B // XCHUNK, XCHUNK)
    nrow = (B // XCHUNK) // NW
    mesh = plsc.VectorSubcoreMesh(core_axis_name="c", subcore_axis_name="s")
    call = pl.kernel(
        _sc_gather, mesh=mesh,
        compiler_params=pltpu.CompilerParams(use_tc_tiling_on_sc=False),
        out_type=[
            jax.ShapeDtypeStruct((B, NODES, IN_DIM), jnp.float32),
            jax.ShapeDtypeStruct((B, NODES, NODES), jnp.float32),
        ],
        scratch_types=[
            pltpu.VMEM((nrow, XCHUNK), jnp.int32),
            pltpu.VMEM((XCHUNK, NODES, IN_DIM), jnp.float32),
            pltpu.VMEM((XCHUNK, NODES, NODES), jnp.float32),
            pltpu.SemaphoreType.DMA,
        ],
    )
    return call(graph_x, kernel_t, idx2d)


def _tc_kernel(x_ref, k_ref, w0_ref, w1_ref, wlin_ref, out_ref,
               kbd, m1s, dall, asel):
    @pl.when(pl.program_id(0) == 0)
    def _init():
        kbd[...] = jnp.zeros((GP, GP), jnp.float32)
        m1s[...] = jnp.zeros((GP, H0), jnp.float32)
        dall[...] = jnp.zeros((GP, 1), jnp.float32)
        asel[...] = jnp.zeros((G, GP), jnp.float32)

    for g in range(G):
        k = k_ref[g]
        dinv_r = jax.lax.rsqrt(jnp.sum(k, axis=1, keepdims=True))
        dinv_c = jax.lax.rsqrt(jnp.sum(k, axis=0, keepdims=True))
        m1 = jnp.dot(x_ref[g], w0_ref[...], preferred_element_type=jnp.float32)
        m1s[g * P:g * P + NODES, :] = dinv_r * m1
        dall[g * P:g * P + NODES, :] = dinv_r
        kbd[g * P:g * P + NODES, g * P:g * P + NODES] = k
        asel[g:g + 1, g * P:g * P + NODES] = k[0:1, :] * dinv_c * dinv_r[0:1, :]

    d = dall[...]
    hag = jnp.dot(kbd[...], m1s[...], preferred_element_type=jnp.float32,
                  precision=jax.lax.Precision.HIGHEST)
    h = jnp.maximum(hag * d, 0.0)
    m2 = jnp.dot(h, w1_ref[...], preferred_element_type=jnp.float32)
    w2 = jnp.dot(asel[...], m2, preferred_element_type=jnp.float32,
                 precision=jax.lax.Precision.HIGHEST)
    z = jnp.maximum(w2, 0.0)
    p = jnp.maximum(jnp.dot(z, wlin_ref[...],
                            preferred_element_type=jnp.float32), 0.0)
    out_ref[...] = p.reshape(G, 1, 1)


def kernel(indices, graph_x, kernel, W0, W1, Wlin):
    B = indices.shape[0]
    xg, kg = _sc_gather_call(graph_x, kernel, indices)
    out = pl.pallas_call(
        _tc_kernel,
        grid=(B // G,),
        in_specs=[
            pl.BlockSpec((G, NODES, IN_DIM), lambda i: (i, 0, 0)),
            pl.BlockSpec((G, NODES, NODES), lambda i: (i, 0, 0)),
            pl.BlockSpec((IN_DIM, H0), lambda i: (0, 0)),
            pl.BlockSpec((H0, H1), lambda i: (0, 0)),
            pl.BlockSpec((H1, 1), lambda i: (0, 0)),
        ],
        out_specs=pl.BlockSpec((G, 1, 1), lambda i: (i, 0, 0)),
        scratch_shapes=[
            pltpu.VMEM((GP, GP), jnp.float32),
            pltpu.VMEM((GP, H0), jnp.float32),
            pltpu.VMEM((GP, 1), jnp.float32),
            pltpu.VMEM((G, GP), jnp.float32),
        ],
        out_shape=jax.ShapeDtypeStruct((B, 1, 1), jnp.float32),
    )(xg, kg, W0, W1, Wlin)
    return out.reshape(B, 1)


# speedup vs baseline: 1.4205x; 1.4205x over previous
import jax
import jax.numpy as jnp
from jax.experimental import pallas as pl
from jax.experimental.pallas import tpu as pltpu

NODES = 26
P = 32
IN_DIM = 128
H0 = 48
H1 = 60
G = 8
NH = 2
GT = G * NH
GP = G * P


def _graph_kernel(idx_ref, *refs):
    x_refs = refs[:GT]
    k_refs = refs[GT:2 * GT]
    w0_ref, w1_ref, wlin_ref, out_ref, kbd, m1s, dall, asel = refs[2 * GT:]

    @pl.when(pl.program_id(0) == 0)
    def _init():
        kbd[...] = jnp.zeros((NH, GP, GP), jnp.float32)
        m1s[...] = jnp.zeros((NH, GP, H0), jnp.float32)
        dall[...] = jnp.zeros((NH, GP, 1), jnp.float32)
        asel[...] = jnp.zeros((NH, G, GP), jnp.float32)

    for h in range(NH):
        for g in range(G):
            x = x_refs[h * G + g][0]
            k = k_refs[h * G + g][0]
            dinv_r = jax.lax.rsqrt(jnp.sum(k, axis=1, keepdims=True))
            dinv_c = jax.lax.rsqrt(jnp.sum(k, axis=0, keepdims=True))
            m1 = jnp.dot(x, w0_ref[...], preferred_element_type=jnp.float32)
            m1s[h, g * P:g * P + NODES, :] = dinv_r * m1
            dall[h, g * P:g * P + NODES, :] = dinv_r
            kbd[h, g * P:g * P + NODES, g * P:g * P + NODES] = k
            asel[h, g:g + 1, g * P:g * P + NODES] = (
                k[0:1, :] * dinv_c * dinv_r[0:1, :])

    ps = []
    for h in range(NH):
        hag = jnp.dot(kbd[h], m1s[h], preferred_element_type=jnp.float32,
                      precision=jax.lax.Precision.HIGHEST)
        hh = jnp.maximum(hag * dall[h], 0.0)
        m2 = jnp.dot(hh, w1_ref[...], preferred_element_type=jnp.float32)
        w2 = jnp.dot(asel[h], m2, preferred_element_type=jnp.float32,
                     precision=jax.lax.Precision.HIGHEST)
        z = jnp.maximum(w2, 0.0)
        ps.append(jnp.maximum(
            jnp.dot(z, wlin_ref[...],
                    preferred_element_type=jnp.float32), 0.0))
    out_ref[...] = jnp.concatenate(ps, axis=0).reshape(GT, 1, 1)


def kernel(indices, graph_x, kernel, W0, W1, Wlin):
    B = indices.shape[0]
    x_specs = [
        pl.BlockSpec((1, NODES, IN_DIM),
                     (lambda i, idx, g=g: (idx[i * GT + g], 0, 0)))
        for g in range(GT)
    ]
    k_specs = [
        pl.BlockSpec((1, NODES, NODES),
                     (lambda i, idx, g=g: (idx[i * GT + g], 0, 0)))
        for g in range(GT)
    ]
    grid_spec = pltpu.PrefetchScalarGridSpec(
        num_scalar_prefetch=1,
        grid=(B // GT,),
        in_specs=x_specs + k_specs + [
            pl.BlockSpec((IN_DIM, H0), lambda i, idx: (0, 0)),
            pl.BlockSpec((H0, H1), lambda i, idx: (0, 0)),
            pl.BlockSpec((H1, 1), lambda i, idx: (0, 0)),
        ],
        out_specs=pl.BlockSpec((GT, 1, 1), lambda i, idx: (i, 0, 0)),
        scratch_shapes=[
            pltpu.VMEM((NH, GP, GP), jnp.float32),
            pltpu.VMEM((NH, GP, H0), jnp.float32),
            pltpu.VMEM((NH, GP, 1), jnp.float32),
            pltpu.VMEM((NH, G, GP), jnp.float32),
        ],
    )
    out = pl.pallas_call(
        _graph_kernel,
        grid_spec=grid_spec,
        out_shape=jax.ShapeDtypeStruct((B, 1, 1), jnp.float32),
    )(indices, *([graph_x] * GT), *([kernel] * GT), W0, W1, Wlin)
    return out.reshape(B, 1)


# speedup vs baseline: 1.4356x; 1.0106x over previous
import jax
import jax.numpy as jnp
from jax.experimental import pallas as pl
from jax.experimental.pallas import tpu as pltpu

NODES = 26
P = 32
IN_DIM = 128
H0 = 48
H1 = 60
G = 8
GP = G * P


def _graph_kernel(idx_ref, *refs):
    x_refs = refs[:G]
    k_refs = refs[G:2 * G]
    w0_ref, w1_ref, wlin_ref, out_ref, kbd, m1s, dall, asel = refs[2 * G:]

    @pl.when(pl.program_id(0) == 0)
    def _init():
        kbd[...] = jnp.zeros((GP, GP), jnp.float32)
        m1s[...] = jnp.zeros((GP, H0), jnp.float32)
        dall[...] = jnp.zeros((GP, 1), jnp.float32)
        asel[...] = jnp.zeros((G, GP), jnp.float32)

    for g in range(G):
        x = x_refs[g][0]
        k = k_refs[g][0]
        dinv_r = jax.lax.rsqrt(jnp.sum(k, axis=1, keepdims=True))
        dinv_c = jax.lax.rsqrt(jnp.sum(k, axis=0, keepdims=True))
        m1 = jnp.dot(x, w0_ref[...], preferred_element_type=jnp.float32)
        m1s[g * P:g * P + NODES, :] = dinv_r * m1
        dall[g * P:g * P + NODES, :] = dinv_r
        kbd[g * P:g * P + NODES, g * P:g * P + NODES] = k
        asel[g:g + 1, g * P:g * P + NODES] = k[0:1, :] * dinv_c * dinv_r[0:1, :]

    hag = jnp.dot(kbd[...], m1s[...], preferred_element_type=jnp.float32,
                  precision=jax.lax.Precision.HIGHEST)
    h = jnp.maximum(hag * dall[...], 0.0)
    m2 = jnp.dot(h, w1_ref[...], preferred_element_type=jnp.float32)
    w2 = jnp.dot(asel[...], m2, preferred_element_type=jnp.float32,
                 precision=jax.lax.Precision.HIGHEST)
    z = jnp.maximum(w2, 0.0)
    p = jnp.maximum(jnp.dot(z, wlin_ref[...],
                            preferred_element_type=jnp.float32), 0.0)
    out_ref[...] = p.reshape(G, 1, 1)


def kernel(indices, graph_x, kernel, W0, W1, Wlin):
    B = indices.shape[0]
    x_specs = [
        pl.BlockSpec((1, NODES, IN_DIM),
                     (lambda i, idx, g=g: (idx[i * G + g], 0, 0)))
        for g in range(G)
    ]
    k_specs = [
        pl.BlockSpec((1, NODES, NODES),
                     (lambda i, idx, g=g: (idx[i * G + g], 0, 0)))
        for g in range(G)
    ]
    grid_spec = pltpu.PrefetchScalarGridSpec(
        num_scalar_prefetch=1,
        grid=(B // G,),
        in_specs=x_specs + k_specs + [
            pl.BlockSpec((IN_DIM, H0), lambda i, idx: (0, 0)),
            pl.BlockSpec((H0, H1), lambda i, idx: (0, 0)),
            pl.BlockSpec((H1, 1), lambda i, idx: (0, 0)),
        ],
        out_specs=pl.BlockSpec((G, 1, 1), lambda i, idx: (i, 0, 0)),
        scratch_shapes=[
            pltpu.VMEM((GP, GP), jnp.float32),
            pltpu.VMEM((GP, H0), jnp.float32),
            pltpu.VMEM((GP, 1), jnp.float32),
            pltpu.VMEM((G, GP), jnp.float32),
        ],
    )
    out = pl.pallas_call(
        _graph_kernel,
        grid_spec=grid_spec,
        out_shape=jax.ShapeDtypeStruct((B, 1, 1), jnp.float32),
    )(indices, *([graph_x] * G), *([kernel] * G), W0, W1, Wlin)
    return out.reshape(B, 1)
